# Initial kernel scaffold; baseline (speedup 1.0000x reference)
import jax
import jax.numpy as jnp
from jax.experimental import pallas as pl


def kernel(sp_center_feat, sp_center_coord, rawPoint_feat, hilbert_feat_coord, points_feat, points_coord, level0_to_level1_indices):
    raise NotImplementedError("write your pallas kernel here")


# speedup vs baseline: 3.9724x; 3.9724x over previous
import jax
import jax.numpy as jnp
from jax.experimental import pallas as pl
from jax.experimental.pallas import tpu as pltpu

NS0 = 1024
PPS0 = 64
C = 256
M = 2048

_INTERPRET = False


def _assign_body(l2l_ref, pf_ref, gm_ref, g0_ref, gp_ref, hb_ref, asg_ref, par_ref):
    i = pl.program_id(0)
    x = pf_ref[0]
    G3 = jnp.concatenate([gm_ref[0], g0_ref[0], gp_ref[0]], axis=0)
    sims = jax.lax.dot_general(
        x, G3, (((1,), (1,)), ((), ())), preferred_element_type=jnp.float32,
        precision=jax.lax.Precision.HIGHEST,
    )
    s0 = sims[:, 0:1]
    s1 = sims[:, 1:2]
    s2 = sims[:, 2:3]
    b01 = s0 >= s1
    v01 = jnp.where(b01, s0, s1)
    w2 = s2 > v01
    id_m = l2l_ref[jnp.maximum(i - 1, 0)]
    id_0 = l2l_ref[i]
    id_p = l2l_ref[jnp.minimum(i + 1, NS0 - 1)]
    assigned = jnp.where(w2, id_p, jnp.where(b01, id_m, id_0))
    asg_ref[0] = assigned

    win = jnp.where(w2, 2, jnp.where(b01, 0, 1))
    hb = hb_ref[0]
    lane = jax.lax.broadcasted_iota(jnp.int32, (1, 16), 1)
    row = jnp.zeros((1, 16), jnp.float32)
    for k, idk in enumerate((id_m, id_0, id_p)):
        sel = (win == k).astype(jnp.float32)
        cnt = jnp.sum(sel)
        row = row + jnp.where(lane == 5 * k, idk.astype(jnp.float32), 0.0)
        row = row + jnp.where(lane == 5 * k + 1, cnt, 0.0)
        for d in range(3):
            sv = jnp.sum(sel * hb[:, d : d + 1])
            row = row + jnp.where(lane == 5 * k + 2 + d, sv, 0.0)
    par_ref[0] = row


def _scatter_body(p_ref, o_ref):
    m = pl.program_id(0)
    P = p_ref[...]
    base = m * 128
    colf = (base + jax.lax.broadcasted_iota(jnp.int32, (1, 128), 1)).astype(jnp.float32)
    acc = jnp.zeros((128, 4), jnp.float32)
    for k in range(3):
        idk = jax.lax.slice(P, (0, 5 * k), (NS0, 5 * k + 1))
        vals = jax.lax.slice(P, (0, 5 * k + 1), (NS0, 5 * k + 5))
        mask = (idk == colf).astype(jnp.float32)
        acc = acc + jax.lax.dot_general(
            mask, vals, (((0,), (0,)), ((), ())), preferred_element_type=jnp.float32,
            precision=jax.lax.Precision.HIGHEST,
        )
    coord = acc[:, 1:4] / jnp.maximum(acc[:, 0:1], 1.0)
    o_ref[...] = coord


def kernel(sp_center_feat, sp_center_coord, rawPoint_feat, hilbert_feat_coord,
           points_feat, points_coord, level0_to_level1_indices):
    del sp_center_coord, points_coord
    l2l = level0_to_level1_indices.astype(jnp.int32)
    hb3 = hilbert_feat_coord.reshape(NS0, PPS0, 3)
    sp3 = sp_center_feat.reshape(M, 1, C)

    grid_spec = pltpu.PrefetchScalarGridSpec(
        num_scalar_prefetch=1,
        grid=(NS0,),
        in_specs=[
            pl.BlockSpec((1, PPS0, C), lambda i, l2l_ref: (i, 0, 0)),
            pl.BlockSpec((1, 1, C), lambda i, l2l_ref: (l2l_ref[jnp.maximum(i - 1, 0)], 0, 0)),
            pl.BlockSpec((1, 1, C), lambda i, l2l_ref: (l2l_ref[i], 0, 0)),
            pl.BlockSpec((1, 1, C), lambda i, l2l_ref: (l2l_ref[jnp.minimum(i + 1, NS0 - 1)], 0, 0)),
            pl.BlockSpec((1, PPS0, 3), lambda i, l2l_ref: (i, 0, 0)),
        ],
        out_specs=[
            pl.BlockSpec((1, PPS0, 1), lambda i, l2l_ref: (i, 0, 0)),
            pl.BlockSpec((1, 1, 16), lambda i, l2l_ref: (i, 0, 0)),
        ],
    )
    asg, par = pl.pallas_call(
        _assign_body,
        grid_spec=grid_spec,
        out_shape=[
            jax.ShapeDtypeStruct((NS0, PPS0, 1), jnp.int32),
            jax.ShapeDtypeStruct((NS0, 1, 16), jnp.float32),
        ],
        interpret=_INTERPRET,
    )(l2l, points_feat, sp3, sp3, sp3, hb3)

    new_coord = pl.pallas_call(
        _scatter_body,
        grid=(M // 128,),
        in_specs=[pl.BlockSpec((NS0, 16), lambda m: (0, 0))],
        out_specs=pl.BlockSpec((128, 3), lambda m: (m, 0)),
        out_shape=jax.ShapeDtypeStruct((M, 3), jnp.float32),
        interpret=_INTERPRET,
    )(par.reshape(NS0, 16))

    point_assignments = asg.reshape(-1)
    points_feat_out = rawPoint_feat.reshape(NS0, PPS0, C)
    hilbert_feat_level1 = rawPoint_feat.reshape(M, 32, C)
    return (point_assignments, sp_center_feat, new_coord, points_feat_out,
            hilbert_feat_level1)


# speedup vs baseline: 11.7881x; 2.9675x over previous
import jax
import jax.numpy as jnp
from jax.experimental import pallas as pl
from jax.experimental.pallas import tpu as pltpu

NS0 = 1024
PPS0 = 64
C = 256
M = 2048
SEG = 8
NBLK = NS0 // SEG
PTS = SEG * PPS0
BAND = SEG + 2
NPAR = SEG * 3

_INTERPRET = False
_HI = jax.lax.Precision.HIGHEST


def _assign_body(l2l_ref, pf_ref, hb_ref, *refs):
    g_refs = refs[:BAND]
    asg_ref, par_ref = refs[BAND], refs[BAND + 1]
    b = pl.program_id(0)
    x = pf_ref[...].reshape(PTS, C)
    band = jnp.concatenate([g[0] for g in g_refs], axis=0)
    sims = jax.lax.dot_general(
        x, band, (((1,), (1,)), ((), ())), preferred_element_type=jnp.float32,
        precision=_HI,
    )

    jcol = jax.lax.broadcasted_iota(jnp.int32, (PTS, 1), 0) // PPS0
    col10 = jax.lax.broadcasted_iota(jnp.int32, (PTS, BAND), 1)
    s = []
    for k in range(3):
        mk = col10 == (jcol + k)
        s.append(jnp.sum(jnp.where(mk, sims, 0.0), axis=1, keepdims=True))
    b01 = s[0] >= s[1]
    v01 = jnp.where(b01, s[0], s[1])
    w2 = s[2] > v01
    w = jnp.where(w2, 2, jnp.where(b01, 0, 1))

    idv = [l2l_ref[jnp.clip(SEG * b - 1 + t, 0, NS0 - 1)] for t in range(BAND)]
    t_win = jcol + w
    assigned = jnp.zeros((PTS, 1), jnp.int32)
    for t in range(BAND):
        assigned = assigned + jnp.where(t_win == t, idv[t], 0)
    asg_ref[...] = assigned

    hb = hb_ref[...].reshape(PTS, 3)
    c24 = jax.lax.broadcasted_iota(jnp.int32, (PTS, NPAR), 1)
    m24 = ((jcol == c24 // 3) & (w == c24 % 3)).astype(jnp.float32)
    rhs = jnp.concatenate([jnp.ones((PTS, 1), jnp.float32), hb], axis=1)
    out24 = jax.lax.dot_general(
        m24, rhs, (((0,), (0,)), ((), ())), preferred_element_type=jnp.float32,
        precision=_HI,
    )
    rio = jax.lax.broadcasted_iota(jnp.int32, (NPAR, 1), 0)
    tval = rio // 3 + rio % 3
    idcol = jnp.zeros((NPAR, 1), jnp.float32)
    for t in range(BAND):
        idcol = idcol + jnp.where(tval == t, jnp.float32(1.0) * idv[t], 0.0)
    par_ref[0] = jnp.concatenate([idcol, out24], axis=1)


def _scatter_body(p_ref, o_ref):
    m = pl.program_id(0)
    P = p_ref[...]
    colf = (m * 128 + jax.lax.broadcasted_iota(jnp.int32, (1, 128), 1)).astype(jnp.float32)
    idc = jax.lax.slice(P, (0, 0), (NS0 * 3, 1))
    vals = jax.lax.slice(P, (0, 1), (NS0 * 3, 5))
    mask = (idc == colf).astype(jnp.float32)
    acc = jax.lax.dot_general(
        mask, vals, (((0,), (0,)), ((), ())), preferred_element_type=jnp.float32,
        precision=_HI,
    )
    coord = acc[:, 1:4] / jnp.maximum(acc[:, 0:1], 1.0)
    o_ref[...] = coord


def _copy_body(in_ref, o1_ref, o2_ref):
    x = in_ref[...]
    o1_ref[...] = x.reshape(32, PPS0, C)
    o2_ref[...] = x.reshape(64, 32, C)


def kernel(sp_center_feat, sp_center_coord, rawPoint_feat, hilbert_feat_coord,
           points_feat, points_coord, level0_to_level1_indices):
    del sp_center_coord, points_coord
    l2l = level0_to_level1_indices.astype(jnp.int32)
    hb3 = hilbert_feat_coord.reshape(NS0, PPS0, 3)
    sp3 = sp_center_feat.reshape(M, 1, C)

    def _g_spec(t):
        return pl.BlockSpec(
            (1, 1, C),
            lambda bb, l2l_ref, _t=t: (l2l_ref[jnp.clip(SEG * bb - 1 + _t, 0, NS0 - 1)], 0, 0),
        )

    grid_spec = pltpu.PrefetchScalarGridSpec(
        num_scalar_prefetch=1,
        grid=(NBLK,),
        in_specs=[
            pl.BlockSpec((SEG, PPS0, C), lambda bb, l2l_ref: (bb, 0, 0)),
            pl.BlockSpec((SEG, PPS0, 3), lambda bb, l2l_ref: (bb, 0, 0)),
        ] + [_g_spec(t) for t in range(BAND)],
        out_specs=[
            pl.BlockSpec((PTS, 1), lambda bb, l2l_ref: (bb, 0)),
            pl.BlockSpec((1, NPAR, 5), lambda bb, l2l_ref: (bb, 0, 0)),
        ],
    )
    asg, par = pl.pallas_call(
        _assign_body,
        grid_spec=grid_spec,
        out_shape=[
            jax.ShapeDtypeStruct((NS0 * PPS0, 1), jnp.int32),
            jax.ShapeDtypeStruct((NBLK, NPAR, 5), jnp.float32),
        ],
        interpret=_INTERPRET,
    )(l2l, points_feat, hb3, *([sp3] * BAND))

    new_coord = pl.pallas_call(
        _scatter_body,
        grid=(M // 128,),
        in_specs=[pl.BlockSpec((NS0 * 3, 5), lambda m: (0, 0))],
        out_specs=pl.BlockSpec((128, 3), lambda m: (m, 0)),
        out_shape=jax.ShapeDtypeStruct((M, 3), jnp.float32),
        interpret=_INTERPRET,
    )(par.reshape(NS0 * 3, 5))

    raw2 = rawPoint_feat
    points_feat_out, hilbert_feat_level1 = pl.pallas_call(
        _copy_body,
        grid=(32,),
        in_specs=[pl.BlockSpec((2048, C), lambda b: (b, 0))],
        out_specs=[
            pl.BlockSpec((32, PPS0, C), lambda b: (b, 0, 0)),
            pl.BlockSpec((64, 32, C), lambda b: (b, 0, 0)),
        ],
        out_shape=[
            jax.ShapeDtypeStruct((NS0, PPS0, C), jnp.float32),
            jax.ShapeDtypeStruct((M, 32, C), jnp.float32),
        ],
        interpret=_INTERPRET,
    )(raw2)

    point_assignments = asg.reshape(-1)
    return (point_assignments, sp_center_feat, new_coord, points_feat_out,
            hilbert_feat_level1)


# speedup vs baseline: 13.9903x; 1.1868x over previous
import jax
import jax.numpy as jnp
from jax.experimental import pallas as pl
from jax.experimental.pallas import tpu as pltpu

NS0 = 1024
PPS0 = 64
C = 256
M = 2048
SEG = 8
NBLK = NS0 // SEG
PTS = SEG * PPS0
BAND = SEG + 2
NPAR = SEG * 3

_INTERPRET = False
_HI = jax.lax.Precision.HIGHEST


def _assign_body(l2l_ref, pf_ref, hb_ref, raw_ref, *refs):
    g_refs = refs[:BAND]
    asg_ref, par_ref, o1_ref, o2_ref = refs[BAND:BAND + 4]
    b = pl.program_id(0)
    xraw = raw_ref[...]
    o1_ref[...] = xraw.reshape(SEG, PPS0, C)
    o2_ref[...] = xraw.reshape(SEG * 2, 32, C)
    x = pf_ref[...].reshape(PTS, C)
    band = jnp.concatenate([g[0] for g in g_refs], axis=0)
    sims = jax.lax.dot_general(
        x, band, (((1,), (1,)), ((), ())), preferred_element_type=jnp.float32,
        precision=_HI,
    )

    jcol = jax.lax.broadcasted_iota(jnp.int32, (PTS, 1), 0) // PPS0
    col10 = jax.lax.broadcasted_iota(jnp.int32, (PTS, BAND), 1)
    s = []
    for k in range(3):
        mk = col10 == (jcol + k)
        s.append(jnp.sum(jnp.where(mk, sims, 0.0), axis=1, keepdims=True))
    b01 = s[0] >= s[1]
    v01 = jnp.where(b01, s[0], s[1])
    w2 = s[2] > v01
    w = jnp.where(w2, 2, jnp.where(b01, 0, 1))

    idv = [l2l_ref[jnp.clip(SEG * b - 1 + t, 0, NS0 - 1)] for t in range(BAND)]
    t_win = jcol + w
    assigned = jnp.zeros((PTS, 1), jnp.int32)
    for t in range(BAND):
        assigned = assigned + jnp.where(t_win == t, idv[t], 0)
    asg_ref[...] = assigned

    hb = hb_ref[...].reshape(PTS, 3)
    c24 = jax.lax.broadcasted_iota(jnp.int32, (PTS, NPAR), 1)
    m24 = ((jcol == c24 // 3) & (w == c24 % 3)).astype(jnp.float32)
    rhs = jnp.concatenate([jnp.ones((PTS, 1), jnp.float32), hb], axis=1)
    out24 = jax.lax.dot_general(
        m24, rhs, (((0,), (0,)), ((), ())), preferred_element_type=jnp.float32,
        precision=_HI,
    )
    rio = jax.lax.broadcasted_iota(jnp.int32, (NPAR, 1), 0)
    tval = rio // 3 + rio % 3
    idcol = jnp.zeros((NPAR, 1), jnp.float32)
    for t in range(BAND):
        idcol = idcol + jnp.where(tval == t, jnp.float32(1.0) * idv[t], 0.0)
    par_ref[0] = jnp.concatenate([idcol, out24], axis=1)


def _scatter_body(p_ref, o_ref):
    m = pl.program_id(0)
    P = p_ref[...]
    colf = (m * 128 + jax.lax.broadcasted_iota(jnp.int32, (1, 128), 1)).astype(jnp.float32)
    idc = jax.lax.slice(P, (0, 0), (NS0 * 3, 1))
    vals = jax.lax.slice(P, (0, 1), (NS0 * 3, 5))
    mask = (idc == colf).astype(jnp.float32)
    acc = jax.lax.dot_general(
        mask, vals, (((0,), (0,)), ((), ())), preferred_element_type=jnp.float32,
        precision=_HI,
    )
    coord = acc[:, 1:4] / jnp.maximum(acc[:, 0:1], 1.0)
    o_ref[...] = coord


def kernel(sp_center_feat, sp_center_coord, rawPoint_feat, hilbert_feat_coord,
           points_feat, points_coord, level0_to_level1_indices):
    del sp_center_coord, points_coord
    l2l = level0_to_level1_indices.astype(jnp.int32)
    hb3 = hilbert_feat_coord.reshape(NS0, PPS0, 3)
    sp3 = sp_center_feat.reshape(M, 1, C)

    def _g_spec(t):
        return pl.BlockSpec(
            (1, 1, C),
            lambda bb, l2l_ref, _t=t: (l2l_ref[jnp.clip(SEG * bb - 1 + _t, 0, NS0 - 1)], 0, 0),
        )

    grid_spec = pltpu.PrefetchScalarGridSpec(
        num_scalar_prefetch=1,
        grid=(NBLK,),
        in_specs=[
            pl.BlockSpec((SEG, PPS0, C), lambda bb, l2l_ref: (bb, 0, 0)),
            pl.BlockSpec((SEG, PPS0, 3), lambda bb, l2l_ref: (bb, 0, 0)),
            pl.BlockSpec((PTS, C), lambda bb, l2l_ref: (bb, 0)),
        ] + [_g_spec(t) for t in range(BAND)],
        out_specs=[
            pl.BlockSpec((PTS, 1), lambda bb, l2l_ref: (bb, 0)),
            pl.BlockSpec((1, NPAR, 5), lambda bb, l2l_ref: (bb, 0, 0)),
            pl.BlockSpec((SEG, PPS0, C), lambda bb, l2l_ref: (bb, 0, 0)),
            pl.BlockSpec((SEG * 2, 32, C), lambda bb, l2l_ref: (bb, 0, 0)),
        ],
    )
    asg, par, points_feat_out, hilbert_feat_level1 = pl.pallas_call(
        _assign_body,
        grid_spec=grid_spec,
        out_shape=[
            jax.ShapeDtypeStruct((NS0 * PPS0, 1), jnp.int32),
            jax.ShapeDtypeStruct((NBLK, NPAR, 5), jnp.float32),
            jax.ShapeDtypeStruct((NS0, PPS0, C), jnp.float32),
            jax.ShapeDtypeStruct((M, 32, C), jnp.float32),
        ],
        interpret=_INTERPRET,
    )(l2l, points_feat, hb3, rawPoint_feat, *([sp3] * BAND))

    new_coord = pl.pallas_call(
        _scatter_body,
        grid=(M // 128,),
        in_specs=[pl.BlockSpec((NS0 * 3, 5), lambda m: (0, 0))],
        out_specs=pl.BlockSpec((128, 3), lambda m: (m, 0)),
        out_shape=jax.ShapeDtypeStruct((M, 3), jnp.float32),
        interpret=_INTERPRET,
    )(par.reshape(NS0 * 3, 5))

    point_assignments = asg.reshape(-1)
    return (point_assignments, sp_center_feat, new_coord, points_feat_out,
            hilbert_feat_level1)


# speedup vs baseline: 14.2431x; 1.0181x over previous
import jax
import jax.numpy as jnp
from jax.experimental import pallas as pl
from jax.experimental.pallas import tpu as pltpu

NS0 = 1024
PPS0 = 64
C = 256
M = 2048
SEG = 16
NBLK = NS0 // SEG
PTS = SEG * PPS0
BAND = SEG + 2
NPAR = SEG * 3

_INTERPRET = False
_HI = jax.lax.Precision.HIGHEST


def _assign_body(l2l_ref, pf_ref, hb_ref, raw_ref, *refs):
    g_refs = refs[:BAND]
    asg_ref, par_ref, o1_ref, o2_ref = refs[BAND:BAND + 4]
    b = pl.program_id(0)
    xraw = raw_ref[...]
    o1_ref[...] = xraw.reshape(SEG, PPS0, C)
    o2_ref[...] = xraw.reshape(SEG * 2, 32, C)
    x = pf_ref[...].reshape(PTS, C)
    band = jnp.concatenate([g[0] for g in g_refs], axis=0)
    sims = jax.lax.dot_general(
        x, band, (((1,), (1,)), ((), ())), preferred_element_type=jnp.float32,
        precision=_HI,
    )

    jcol = jax.lax.broadcasted_iota(jnp.int32, (PTS, 1), 0) // PPS0
    col10 = jax.lax.broadcasted_iota(jnp.int32, (PTS, BAND), 1)
    s = []
    for k in range(3):
        mk = col10 == (jcol + k)
        s.append(jnp.sum(jnp.where(mk, sims, 0.0), axis=1, keepdims=True))
    b01 = s[0] >= s[1]
    v01 = jnp.where(b01, s[0], s[1])
    w2 = s[2] > v01
    w = jnp.where(w2, 2, jnp.where(b01, 0, 1))

    idv = [l2l_ref[jnp.clip(SEG * b - 1 + t, 0, NS0 - 1)] for t in range(BAND)]
    t_win = jcol + w
    assigned = jnp.zeros((PTS, 1), jnp.int32)
    for t in range(BAND):
        assigned = assigned + jnp.where(t_win == t, idv[t], 0)
    asg_ref[...] = assigned

    hb = hb_ref[...].reshape(PTS, 3)
    c24 = jax.lax.broadcasted_iota(jnp.int32, (PTS, NPAR), 1)
    m24 = ((jcol == c24 // 3) & (w == c24 % 3)).astype(jnp.float32)
    rhs = jnp.concatenate([jnp.ones((PTS, 1), jnp.float32), hb], axis=1)
    out24 = jax.lax.dot_general(
        m24, rhs, (((0,), (0,)), ((), ())), preferred_element_type=jnp.float32,
        precision=_HI,
    )
    rio = jax.lax.broadcasted_iota(jnp.int32, (NPAR, 1), 0)
    tval = rio // 3 + rio % 3
    idcol = jnp.zeros((NPAR, 1), jnp.float32)
    for t in range(BAND):
        idcol = idcol + jnp.where(tval == t, jnp.float32(1.0) * idv[t], 0.0)
    par_ref[0] = jnp.concatenate([idcol, out24], axis=1)


def _scatter_body(p_ref, o_ref):
    m = pl.program_id(0)
    P = p_ref[...]
    colf = (m * 128 + jax.lax.broadcasted_iota(jnp.int32, (1, 128), 1)).astype(jnp.float32)
    idc = jax.lax.slice(P, (0, 0), (NS0 * 3, 1))
    vals = jax.lax.slice(P, (0, 1), (NS0 * 3, 5))
    mask = (idc == colf).astype(jnp.float32)
    acc = jax.lax.dot_general(
        mask, vals, (((0,), (0,)), ((), ())), preferred_element_type=jnp.float32,
        precision=_HI,
    )
    coord = acc[:, 1:4] / jnp.maximum(acc[:, 0:1], 1.0)
    o_ref[...] = coord


def kernel(sp_center_feat, sp_center_coord, rawPoint_feat, hilbert_feat_coord,
           points_feat, points_coord, level0_to_level1_indices):
    del sp_center_coord, points_coord
    l2l = level0_to_level1_indices.astype(jnp.int32)
    hb3 = hilbert_feat_coord.reshape(NS0, PPS0, 3)
    sp3 = sp_center_feat.reshape(M, 1, C)

    def _g_spec(t):
        return pl.BlockSpec(
            (1, 1, C),
            lambda bb, l2l_ref, _t=t: (l2l_ref[jnp.clip(SEG * bb - 1 + _t, 0, NS0 - 1)], 0, 0),
        )

    grid_spec = pltpu.PrefetchScalarGridSpec(
        num_scalar_prefetch=1,
        grid=(NBLK,),
        in_specs=[
            pl.BlockSpec((SEG, PPS0, C), lambda bb, l2l_ref: (bb, 0, 0)),
            pl.BlockSpec((SEG, PPS0, 3), lambda bb, l2l_ref: (bb, 0, 0)),
            pl.BlockSpec((PTS, C), lambda bb, l2l_ref: (bb, 0)),
        ] + [_g_spec(t) for t in range(BAND)],
        out_specs=[
            pl.BlockSpec((PTS, 1), lambda bb, l2l_ref: (bb, 0)),
            pl.BlockSpec((1, NPAR, 5), lambda bb, l2l_ref: (bb, 0, 0)),
            pl.BlockSpec((SEG, PPS0, C), lambda bb, l2l_ref: (bb, 0, 0)),
            pl.BlockSpec((SEG * 2, 32, C), lambda bb, l2l_ref: (bb, 0, 0)),
        ],
    )
    asg, par, points_feat_out, hilbert_feat_level1 = pl.pallas_call(
        _assign_body,
        grid_spec=grid_spec,
        out_shape=[
            jax.ShapeDtypeStruct((NS0 * PPS0, 1), jnp.int32),
            jax.ShapeDtypeStruct((NBLK, NPAR, 5), jnp.float32),
            jax.ShapeDtypeStruct((NS0, PPS0, C), jnp.float32),
            jax.ShapeDtypeStruct((M, 32, C), jnp.float32),
        ],
        interpret=_INTERPRET,
    )(l2l, points_feat, hb3, rawPoint_feat, *([sp3] * BAND))

    new_coord = pl.pallas_call(
        _scatter_body,
        grid=(M // 128,),
        in_specs=[pl.BlockSpec((NS0 * 3, 5), lambda m: (0, 0))],
        out_specs=pl.BlockSpec((128, 3), lambda m: (m, 0)),
        out_shape=jax.ShapeDtypeStruct((M, 3), jnp.float32),
        interpret=_INTERPRET,
    )(par.reshape(NS0 * 3, 5))

    point_assignments = asg.reshape(-1)
    return (point_assignments, sp_center_feat, new_coord, points_feat_out,
            hilbert_feat_level1)
